# speedup vs baseline: 1.7579x; 1.7579x over previous
import functools

import jax
import jax.numpy as jnp
from jax import lax
from jax.experimental import pallas as pl
from jax.experimental.pallas import tpu as pltpu

_VMEM_LIMIT = 64 * 1024 * 1024
_DN = (((1,), (1,)), ((), ()))


def _pick_tile(n, target=256):
    for t in (target, 256, 128, 64, 32, 16, 8):
        if t <= n and n % t == 0:
            return t
    return n


def _resident(shape):
    nd = len(shape)
    return pl.BlockSpec(shape, lambda t: (0,) * nd)


def _params():
    return pltpu.CompilerParams(dimension_semantics=("parallel",),
                                vmem_limit_bytes=_VMEM_LIMIT)


def _gat_tile(x_aug, x_dst, mask, projl_ref, projr_ref, watt_ref, batt_ref,
              *, num_heads, hid):
    el = lax.dot_general(projl_ref[...], x_aug, _DN,
                         preferred_element_type=jnp.float32)
    er = lax.dot_general(x_dst, projr_ref[...], _DN,
                         preferred_element_type=jnp.float32)
    el = jnp.minimum(el, 35.0)
    er = jnp.minimum(er, 35.0)

    parts = []
    for h in range(num_heads):
        e = er[:, h:h + 1] + el[h:h + 1, :]
        e = jnp.maximum(e, 0.2 * e)
        s = jnp.where(mask, e, -1e30)
        p = jnp.exp(s).astype(jnp.bfloat16)
        ua = jnp.dot(p, x_aug, preferred_element_type=jnp.float32)
        denom = jnp.maximum(ua[:, hid:hid + 1], 1e-30)
        parts.append((ua[:, :hid] * pl.reciprocal(denom, approx=True))
                     .astype(jnp.bfloat16))

    slab = jnp.concatenate(parts, axis=-1)
    out = jnp.dot(slab, watt_ref[...], preferred_element_type=jnp.float32)
    return out * (1.0 / num_heads) + batt_ref[...]


def _ones_col(y_bf16, tm):
    return jnp.concatenate(
        [y_bf16, jnp.ones((tm, 1), jnp.bfloat16)], axis=-1)


def _feat_kernel(x_ref, we_ref, be_ref, w1_ref, b1_ref, o_ref, *, tm):
    e = jnp.dot(x_ref[...].astype(jnp.bfloat16), we_ref[...],
                preferred_element_type=jnp.float32) + be_ref[...]
    f = jnp.dot(e.astype(jnp.bfloat16), w1_ref[...],
                preferred_element_type=jnp.float32) + b1_ref[...]
    o_ref[...] = _ones_col(f.astype(jnp.bfloat16), tm)


def _adj0_kernel(feats_ref, adj_ref,
                 p0l_ref, p0r_ref, w0_ref, b0_ref,
                 p1l_ref, p1r_ref, w1_ref, b1_ref,
                 y0_ref, y1_ref, *, num_heads, tm, hid):
    row0 = pl.multiple_of(pl.program_id(0) * tm, tm)
    x_aug = feats_ref[...]
    x_dst = feats_ref[pl.ds(row0, tm), :]
    mask = adj_ref[...] > 0.5
    for (plr, prr, wr, br, yr) in ((p0l_ref, p0r_ref, w0_ref, b0_ref, y0_ref),
                                   (p1l_ref, p1r_ref, w1_ref, b1_ref, y1_ref)):
        y = _gat_tile(x_aug, x_dst, mask, plr, prr, wr, br,
                      num_heads=num_heads, hid=hid)
        yr[...] = _ones_col(y.astype(jnp.bfloat16), tm)


def _lstm_gates(gates, hid):
    i = jax.nn.sigmoid(gates[:, 0 * hid:1 * hid])
    f = jax.nn.sigmoid(gates[:, 1 * hid:2 * hid])
    g = jnp.tanh(gates[:, 2 * hid:3 * hid])
    o = jax.nn.sigmoid(gates[:, 3 * hid:4 * hid])
    return i, f, g, o


def _adj1_kernel(y0_ref, y1_ref, adj_ref,
                 p0l_ref, p0r_ref, w0_ref, b0_ref, wl0_ref, bl0_ref,
                 p1l_ref, p1r_ref, w1_ref, b1_ref, wl1_ref, bl1_ref,
                 pw_ref, pb_ref, o_ref, *, num_heads, tm, hid):
    row0 = pl.multiple_of(pl.program_id(0) * tm, tm)
    mask = adj_ref[...] > 0.5

    g0 = _gat_tile(y0_ref[...], y0_ref[pl.ds(row0, tm), :], mask,
                   p0l_ref, p0r_ref, w0_ref, b0_ref,
                   num_heads=num_heads, hid=hid)
    gates = jnp.dot(g0.astype(jnp.bfloat16), wl0_ref[0:hid, :],
                    preferred_element_type=jnp.float32) + bl0_ref[...]
    i0, _, gg0, o0 = _lstm_gates(gates, hid)
    c = i0 * gg0
    h = o0 * jnp.tanh(c)

    g1 = _gat_tile(y1_ref[...], y1_ref[pl.ds(row0, tm), :], mask,
                   p1l_ref, p1r_ref, w1_ref, b1_ref,
                   num_heads=num_heads, hid=hid)
    xin = jnp.concatenate([g1.astype(jnp.bfloat16), h.astype(jnp.bfloat16)],
                          axis=-1)
    gates = jnp.dot(xin, wl1_ref[...],
                    preferred_element_type=jnp.float32) + bl1_ref[...]
    i1, f1, gg1, o1 = _lstm_gates(gates, hid)
    c = f1 * c + i1 * gg1
    h = o1 * jnp.tanh(c)

    z = jnp.dot(h.astype(jnp.bfloat16), pw_ref[...],
                preferred_element_type=jnp.float32) + pb_ref[...]
    o_ref[...] = jax.nn.sigmoid(z)


def kernel(x, adj0, adj1, embed_w, embed_b, lin1_w, lin1_b, pred_w, pred_b,
           l0_proj_l, l0_proj_r, l0_w_att, l0_b_att, l0_w_lstm, l0_b_lstm,
           l1_proj_l, l1_proj_r, l1_w_att, l1_b_att, l1_w_lstm, l1_b_lstm):
    n = x.shape[0]
    hid = lin1_w.shape[1]
    num_heads = l0_proj_l.shape[0]
    tm = _pick_tile(n, 256)
    n_aug = hid + 1

    zcol = jnp.zeros((num_heads, 1), jnp.bfloat16)
    p0l = jnp.concatenate([l0_proj_l, zcol], axis=-1)
    p0r = jnp.concatenate([l0_proj_r, zcol], axis=-1)
    p1l = jnp.concatenate([l1_proj_l, zcol], axis=-1)
    p1r = jnp.concatenate([l1_proj_r, zcol], axis=-1)

    feats = pl.pallas_call(
        functools.partial(_feat_kernel, tm=tm),
        out_shape=jax.ShapeDtypeStruct((n, n_aug), jnp.bfloat16),
        grid=(n // tm,),
        in_specs=[pl.BlockSpec((tm, x.shape[1]), lambda t: (t, 0)),
                  _resident(embed_w.shape), _resident(embed_b.shape),
                  _resident(lin1_w.shape), _resident(lin1_b.shape)],
        out_specs=pl.BlockSpec((tm, n_aug), lambda t: (t, 0)),
        compiler_params=_params(),
    )(x, embed_w, embed_b, lin1_w, lin1_b)

    y0, y1 = pl.pallas_call(
        functools.partial(_adj0_kernel, num_heads=num_heads, tm=tm, hid=hid),
        out_shape=(jax.ShapeDtypeStruct((n, n_aug), jnp.bfloat16),
                   jax.ShapeDtypeStruct((n, n_aug), jnp.bfloat16)),
        grid=(n // tm,),
        in_specs=[_resident(feats.shape),
                  pl.BlockSpec((tm, n), lambda t: (t, 0)),
                  _resident(p0l.shape), _resident(p0r.shape),
                  _resident(l0_w_att.shape), _resident(l0_b_att.shape),
                  _resident(p1l.shape), _resident(p1r.shape),
                  _resident(l1_w_att.shape), _resident(l1_b_att.shape)],
        out_specs=(pl.BlockSpec((tm, n_aug), lambda t: (t, 0)),
                   pl.BlockSpec((tm, n_aug), lambda t: (t, 0))),
        compiler_params=_params(),
    )(feats, adj0, p0l, p0r, l0_w_att, l0_b_att, p1l, p1r, l1_w_att, l1_b_att)

    return pl.pallas_call(
        functools.partial(_adj1_kernel, num_heads=num_heads, tm=tm, hid=hid),
        out_shape=jax.ShapeDtypeStruct((n, 1), jnp.float32),
        grid=(n // tm,),
        in_specs=[_resident(y0.shape), _resident(y1.shape),
                  pl.BlockSpec((tm, n), lambda t: (t, 0)),
                  _resident(p0l.shape), _resident(p0r.shape),
                  _resident(l0_w_att.shape), _resident(l0_b_att.shape),
                  _resident(l0_w_lstm.shape), _resident(l0_b_lstm.shape),
                  _resident(p1l.shape), _resident(p1r.shape),
                  _resident(l1_w_att.shape), _resident(l1_b_att.shape),
                  _resident(l1_w_lstm.shape), _resident(l1_b_lstm.shape),
                  _resident(pred_w.shape), _resident(pred_b.shape)],
        out_specs=pl.BlockSpec((tm, 1), lambda t: (t, 0)),
        compiler_params=_params(),
    )(y0, y1, adj1,
      p0l, p0r, l0_w_att, l0_b_att, l0_w_lstm, l0_b_lstm,
      p1l, p1r, l1_w_att, l1_b_att, l1_w_lstm, l1_b_lstm,
      pred_w, pred_b)


# speedup vs baseline: 2.1777x; 1.2388x over previous
import functools

import jax
import jax.numpy as jnp
from jax import lax
from jax.experimental import pallas as pl
from jax.experimental.pallas import tpu as pltpu

_VMEM_LIMIT = 64 * 1024 * 1024
_DN = (((1,), (1,)), ((), ()))


def _pick_tile(n, target=256):
    for t in (target, 256, 128, 64, 32, 16, 8):
        if t <= n and n % t == 0:
            return t
    return n


def _resident(shape):
    nd = len(shape)
    return pl.BlockSpec(shape, lambda t: (0,) * nd)


def _params():
    return pltpu.CompilerParams(dimension_semantics=("parallel",),
                                vmem_limit_bytes=_VMEM_LIMIT)


_LOG2E = 1.4426950408889634


def _gat_tile(x_aug, x_dst, adj, projl_ref, projr_ref, watt_ref, batt_ref,
              *, num_heads, hid):
    el = lax.dot_general(projl_ref[...], x_aug, _DN,
                         preferred_element_type=jnp.float32)
    er = lax.dot_general(x_dst, projr_ref[...], _DN,
                         preferred_element_type=jnp.float32)
    el = jnp.minimum(el * _LOG2E, 50.5)
    er = jnp.minimum(er * _LOG2E, 50.5)

    parts = []
    for h in range(num_heads):
        e = er[:, h:h + 1] + el[h:h + 1, :]
        e = jnp.maximum(e, 0.2 * e)
        p = jnp.exp2(e).astype(jnp.bfloat16) * adj
        ua = jnp.dot(p, x_aug, preferred_element_type=jnp.float32)
        denom = jnp.maximum(ua[:, hid:hid + 1], 1e-30)
        parts.append((ua[:, :hid] * pl.reciprocal(denom, approx=True))
                     .astype(jnp.bfloat16))

    slab = jnp.concatenate(parts, axis=-1)
    return jnp.dot(slab, watt_ref[...],
                   preferred_element_type=jnp.float32) + batt_ref[...]


def _ones_col(y_bf16, tm):
    return jnp.concatenate(
        [y_bf16, jnp.ones((tm, 1), jnp.bfloat16)], axis=-1)


def _feat_kernel(x_ref, we_ref, be_ref, w1_ref, b1_ref, o_ref, *, tm):
    e = jnp.dot(x_ref[...].astype(jnp.bfloat16), we_ref[...],
                preferred_element_type=jnp.float32) + be_ref[...]
    f = jnp.dot(e.astype(jnp.bfloat16), w1_ref[...],
                preferred_element_type=jnp.float32) + b1_ref[...]
    o_ref[...] = _ones_col(f.astype(jnp.bfloat16), tm)


def _adj0_kernel(feats_ref, adj_ref,
                 p0l_ref, p0r_ref, w0_ref, b0_ref,
                 p1l_ref, p1r_ref, w1_ref, b1_ref,
                 y0_ref, y1_ref, *, num_heads, tm, hid):
    row0 = pl.multiple_of(pl.program_id(0) * tm, tm)
    x_aug = feats_ref[...]
    x_dst = feats_ref[pl.ds(row0, tm), :]
    adj = adj_ref[...]
    for (plr, prr, wr, br, yr) in ((p0l_ref, p0r_ref, w0_ref, b0_ref, y0_ref),
                                   (p1l_ref, p1r_ref, w1_ref, b1_ref, y1_ref)):
        y = _gat_tile(x_aug, x_dst, adj, plr, prr, wr, br,
                      num_heads=num_heads, hid=hid)
        yr[...] = _ones_col(y.astype(jnp.bfloat16), tm)


def _lstm_gates(gates, hid):
    i = jax.nn.sigmoid(gates[:, 0 * hid:1 * hid])
    f = jax.nn.sigmoid(gates[:, 1 * hid:2 * hid])
    g = jnp.tanh(gates[:, 2 * hid:3 * hid])
    o = jax.nn.sigmoid(gates[:, 3 * hid:4 * hid])
    return i, f, g, o


def _adj1_kernel(y0_ref, y1_ref, adj_ref,
                 p0l_ref, p0r_ref, w0_ref, b0_ref, wl0_ref, bl0_ref,
                 p1l_ref, p1r_ref, w1_ref, b1_ref, wl1_ref, bl1_ref,
                 pw_ref, pb_ref, o_ref, *, num_heads, tm, hid):
    row0 = pl.multiple_of(pl.program_id(0) * tm, tm)
    adj = adj_ref[...]

    g0 = _gat_tile(y0_ref[...], y0_ref[pl.ds(row0, tm), :], adj,
                   p0l_ref, p0r_ref, w0_ref, b0_ref,
                   num_heads=num_heads, hid=hid)
    gates = jnp.dot(g0.astype(jnp.bfloat16), wl0_ref[0:hid, :],
                    preferred_element_type=jnp.float32) + bl0_ref[...]
    i0, _, gg0, o0 = _lstm_gates(gates, hid)
    c = i0 * gg0
    h = o0 * jnp.tanh(c)

    g1 = _gat_tile(y1_ref[...], y1_ref[pl.ds(row0, tm), :], adj,
                   p1l_ref, p1r_ref, w1_ref, b1_ref,
                   num_heads=num_heads, hid=hid)
    xin = jnp.concatenate([g1.astype(jnp.bfloat16), h.astype(jnp.bfloat16)],
                          axis=-1)
    gates = jnp.dot(xin, wl1_ref[...],
                    preferred_element_type=jnp.float32) + bl1_ref[...]
    i1, f1, gg1, o1 = _lstm_gates(gates, hid)
    c = f1 * c + i1 * gg1
    h = o1 * jnp.tanh(c)

    z = jnp.dot(h.astype(jnp.bfloat16), pw_ref[...],
                preferred_element_type=jnp.float32) + pb_ref[...]
    o_ref[...] = jax.nn.sigmoid(z)


def kernel(x, adj0, adj1, embed_w, embed_b, lin1_w, lin1_b, pred_w, pred_b,
           l0_proj_l, l0_proj_r, l0_w_att, l0_b_att, l0_w_lstm, l0_b_lstm,
           l1_proj_l, l1_proj_r, l1_w_att, l1_b_att, l1_w_lstm, l1_b_lstm):
    n = x.shape[0]
    hid = lin1_w.shape[1]
    num_heads = l0_proj_l.shape[0]
    tm = _pick_tile(n, 256)
    n_aug = hid + 1

    zcol = jnp.zeros((num_heads, 1), jnp.bfloat16)
    p0l = jnp.concatenate([l0_proj_l, zcol], axis=-1)
    p0r = jnp.concatenate([l0_proj_r, zcol], axis=-1)
    p1l = jnp.concatenate([l1_proj_l, zcol], axis=-1)
    p1r = jnp.concatenate([l1_proj_r, zcol], axis=-1)
    w0a = (l0_w_att.astype(jnp.float32) * (1.0 / num_heads)).astype(jnp.bfloat16)
    w1a = (l1_w_att.astype(jnp.float32) * (1.0 / num_heads)).astype(jnp.bfloat16)

    feats = pl.pallas_call(
        functools.partial(_feat_kernel, tm=tm),
        out_shape=jax.ShapeDtypeStruct((n, n_aug), jnp.bfloat16),
        grid=(n // tm,),
        in_specs=[pl.BlockSpec((tm, x.shape[1]), lambda t: (t, 0)),
                  _resident(embed_w.shape), _resident(embed_b.shape),
                  _resident(lin1_w.shape), _resident(lin1_b.shape)],
        out_specs=pl.BlockSpec((tm, n_aug), lambda t: (t, 0)),
        compiler_params=_params(),
    )(x, embed_w, embed_b, lin1_w, lin1_b)

    y0, y1 = pl.pallas_call(
        functools.partial(_adj0_kernel, num_heads=num_heads, tm=tm, hid=hid),
        out_shape=(jax.ShapeDtypeStruct((n, n_aug), jnp.bfloat16),
                   jax.ShapeDtypeStruct((n, n_aug), jnp.bfloat16)),
        grid=(n // tm,),
        in_specs=[_resident(feats.shape),
                  pl.BlockSpec((tm, n), lambda t: (t, 0)),
                  _resident(p0l.shape), _resident(p0r.shape),
                  _resident(w0a.shape), _resident(l0_b_att.shape),
                  _resident(p1l.shape), _resident(p1r.shape),
                  _resident(w1a.shape), _resident(l1_b_att.shape)],
        out_specs=(pl.BlockSpec((tm, n_aug), lambda t: (t, 0)),
                   pl.BlockSpec((tm, n_aug), lambda t: (t, 0))),
        compiler_params=_params(),
    )(feats, adj0, p0l, p0r, w0a, l0_b_att, p1l, p1r, w1a, l1_b_att)

    return pl.pallas_call(
        functools.partial(_adj1_kernel, num_heads=num_heads, tm=tm, hid=hid),
        out_shape=jax.ShapeDtypeStruct((n, 1), jnp.float32),
        grid=(n // tm,),
        in_specs=[_resident(y0.shape), _resident(y1.shape),
                  pl.BlockSpec((tm, n), lambda t: (t, 0)),
                  _resident(p0l.shape), _resident(p0r.shape),
                  _resident(w0a.shape), _resident(l0_b_att.shape),
                  _resident(l0_w_lstm.shape), _resident(l0_b_lstm.shape),
                  _resident(p1l.shape), _resident(p1r.shape),
                  _resident(w1a.shape), _resident(l1_b_att.shape),
                  _resident(l1_w_lstm.shape), _resident(l1_b_lstm.shape),
                  _resident(pred_w.shape), _resident(pred_b.shape)],
        out_specs=pl.BlockSpec((tm, 1), lambda t: (t, 0)),
        compiler_params=_params(),
    )(y0, y1, adj1,
      p0l, p0r, w0a, l0_b_att, l0_w_lstm, l0_b_lstm,
      p1l, p1r, w1a, l1_b_att, l1_w_lstm, l1_b_lstm,
      pred_w, pred_b)
